# baseline (speedup 1.0000x reference)
import functools
import math

import jax
import jax.numpy as jnp
import numpy as np
from jax import lax
from jax.experimental import pallas as pl
from jax.experimental.pallas import tpu as pltpu
from jax.experimental.pallas import tpu_sc as plsc

HIDDEN = 384
NHEADS = 8
DH = HIDDEN // NHEADS
CLUSTER_SIZE = 64
L = 4096
NBLK = L // 128
SCALE = 1.0 / math.sqrt(DH)
EPS = 1e-5


def _static_plan():
    n_cluster = max(1, L // CLUSTER_SIZE)
    labels = np.random.RandomState(0).randint(0, n_cluster, size=L)
    index = np.argsort(labels, kind="stable")
    window_sizes = np.bincount(labels).tolist()
    new_sizes = []
    for size in window_sizes:
        if size >= CLUSTER_SIZE * 2:
            num_splits = max(1, size // CLUSTER_SIZE)
            q, r = divmod(size, num_splits)
            new_sizes.extend([q + 1 if i < r else q for i in range(num_splits)])
        else:
            new_sizes.append(size)
    sizes = [s for s in new_sizes if s > 0]
    return index.astype(np.int32), sizes


_PERM_NP, _SIZES = _static_plan()
NWIN = len(_SIZES)

_SEG_NP = np.repeat(np.arange(NWIN, dtype=np.int32), _SIZES)

_SEGQ_NP = _SEG_NP.reshape(NBLK, 128, 1)

_SEGK_NP = np.full((NBLK, 1, 3 * 128), -1, dtype=np.int32)
for _i in range(NBLK):
    _lo = (_i - 1) * 128
    _hi = (_i + 2) * 128
    _s = max(_lo, 0)
    _e = min(_hi, L)
    _SEGK_NP[_i, 0, _s - _lo:_e - _lo] = _SEG_NP[_s:_e]

_SMATT_NP = np.zeros((NBLK, NWIN, 128), dtype=np.float32)
for _i in range(NBLK):
    for _r in range(128):
        _w = _SEG_NP[_i * 128 + _r]
        _SMATT_NP[_i, _w, _r] = 1.0 / _SIZES[_w]

_PERM = jnp.asarray(_PERM_NP)
_SEGQ = jnp.asarray(_SEGQ_NP)
_SEGK = jnp.asarray(_SEGK_NP)
_SMATT = jnp.asarray(_SMATT_NP)


def _nt(a, b):
    return lax.dot_general(a.astype(jnp.bfloat16), b.astype(jnp.bfloat16),
                           (((1,), (1,)), ((), ())),
                           preferred_element_type=jnp.float32)


def _nn(a, b):
    return jnp.dot(a.astype(jnp.bfloat16), b.astype(jnp.bfloat16),
                   preferred_element_type=jnp.float32)


def _sc_gather(x2d, idx):
    rows_per_w = L // 32
    mesh = plsc.VectorSubcoreMesh(core_axis_name="c", subcore_axis_name="s",
                                  num_cores=2, num_subcores=16)

    @functools.partial(
        pl.kernel,
        out_type=jax.ShapeDtypeStruct((L, HIDDEN), jnp.float32),
        mesh=mesh,
        scratch_types=[
            pltpu.VMEM((rows_per_w,), jnp.int32),
            pltpu.VMEM((rows_per_w, HIDDEN), jnp.float32),
            pltpu.SemaphoreType.DMA,
        ],
    )
    def body(x_hbm, idx_hbm, out_hbm, idx_v, rows_v, sem):
        wid = lax.axis_index("s") * 2 + lax.axis_index("c")
        base = wid * rows_per_w
        pltpu.sync_copy(idx_hbm.at[pl.ds(base, rows_per_w)], idx_v)
        pltpu.async_copy(x_hbm.at[idx_v], rows_v, sem).wait()
        pltpu.sync_copy(rows_v, out_hbm.at[pl.ds(base, rows_per_w)])

    return body(x2d, idx)


def _attend(xq, xkv, wq3, wk3, wv3, bq3, bk3, bv3, wot3, mask):
    o_acc = None
    for h in range(NHEADS):
        qh = _nt(xq, wq3[h]) + bq3[h]
        kh = _nt(xkv, wk3[h]) + bk3[h]
        vh = _nt(xkv, wv3[h]) + bv3[h]
        s = _nt(qh, kh) * SCALE
        if mask is not None:
            s = jnp.where(mask, s, -1e30)
        m = jnp.max(s, axis=1, keepdims=True)
        e = jnp.exp(s - m)
        p = e / jnp.sum(e, axis=1, keepdims=True)
        oh = _nn(p, vh)
        contrib = _nn(oh, wot3[h])
        o_acc = contrib if o_acc is None else o_acc + contrib
    return o_acc


def _layernorm(x, g, b):
    mu = jnp.mean(x, axis=1, keepdims=True)
    xc = x - mu
    var = jnp.mean(xc * xc, axis=1, keepdims=True)
    return xc * lax.rsqrt(var + EPS) * g + b


def _local_body(xs_p, xs_c, xs_n, wq3, wk3, wv3, bq3, bk3, bv3, wot3, bo,
                lg, lb, segq, segk, smatt, refined_ref, r_ref, acc_ref):
    i = pl.program_id(0)
    xq = xs_c[...]
    xkv = jnp.concatenate([xs_p[...], xs_c[...], xs_n[...]], axis=0)
    mask = segq[0] == segk[0]
    o = _attend(xq, xkv, wq3[...], wk3[...], wv3[...], bq3[...], bk3[...],
                bv3[...], wot3[...], mask) + bo[...]
    refined = _layernorm(xq + o, lg[...], lb[...])
    refined_ref[...] = refined
    part = jnp.dot(smatt[0], refined, preferred_element_type=jnp.float32)

    @pl.when(i == 0)
    def _():
        acc_ref[...] = part

    @pl.when(i > 0)
    def _():
        acc_ref[...] = acc_ref[...] + part

    @pl.when(i == NBLK - 1)
    def _():
        r_ref[...] = acc_ref[...]


def _global_body(refined, r, wq3, wk3, wv3, bq3, bk3, bv3, wot3, bo, gg, gb,
                 h_ref, vec_ref):
    i = pl.program_id(0)

    @pl.when(i == 0)
    def _():
        rr = r[...]
        o = _attend(rr, rr, wq3[...], wk3[...], wv3[...], bq3[...], bk3[...],
                    bv3[...], wot3[...], None) + bo[...]
        rp = _layernorm(rr + o, gg[...], gb[...])
        vec_ref[...] = jnp.mean(rp, axis=0, keepdims=True)

    h_ref[...] = refined[...] + vec_ref[...]


def _split_heads(Wqkv, bqkv, Wo):
    wq, wk, wv = jnp.split(Wqkv, 3, axis=0)
    wq3 = wq.reshape(NHEADS, DH, HIDDEN)
    wk3 = wk.reshape(NHEADS, DH, HIDDEN)
    wv3 = wv.reshape(NHEADS, DH, HIDDEN)
    bq, bk, bv = jnp.split(bqkv, 3)
    bq3 = bq.reshape(NHEADS, 1, DH)
    bk3 = bk.reshape(NHEADS, 1, DH)
    bv3 = bv.reshape(NHEADS, 1, DH)
    wot3 = Wo.T.reshape(NHEADS, DH, HIDDEN)
    return wq3, wk3, wv3, bq3, bk3, bv3, wot3


def _tc_pipeline(xs2d, l_Wqkv, l_bqkv, l_Wo, l_bo, l_g, l_b,
                 g_Wqkv, g_bqkv, g_Wo, g_bo, g_g, g_b, interpret=False):
    lwq3, lwk3, lwv3, lbq3, lbk3, lbv3, lwot3 = _split_heads(l_Wqkv, l_bqkv, l_Wo)
    gwq3, gwk3, gwv3, gbq3, gbk3, gbv3, gwot3 = _split_heads(g_Wqkv, g_bqkv, g_Wo)
    lbo = l_bo.reshape(1, HIDDEN)
    lg = l_g.reshape(1, HIDDEN)
    lb = l_b.reshape(1, HIDDEN)
    gbo = g_bo.reshape(1, HIDDEN)
    gg = g_g.reshape(1, HIDDEN)
    gb = g_b.reshape(1, HIDDEN)

    full3 = lambda shp: pl.BlockSpec(shp, lambda i: (0,) * len(shp))
    blk = lambda shp, im: pl.BlockSpec(shp, im)

    refined, r = pl.pallas_call(
        _local_body,
        grid=(NBLK,),
        in_specs=[
            blk((128, HIDDEN), lambda i: (jnp.maximum(i - 1, 0), 0)),
            blk((128, HIDDEN), lambda i: (i, 0)),
            blk((128, HIDDEN), lambda i: (jnp.minimum(i + 1, NBLK - 1), 0)),
            full3((NHEADS, DH, HIDDEN)), full3((NHEADS, DH, HIDDEN)),
            full3((NHEADS, DH, HIDDEN)),
            full3((NHEADS, 1, DH)), full3((NHEADS, 1, DH)),
            full3((NHEADS, 1, DH)),
            full3((NHEADS, DH, HIDDEN)),
            full3((1, HIDDEN)), full3((1, HIDDEN)), full3((1, HIDDEN)),
            blk((1, 128, 1), lambda i: (i, 0, 0)),
            blk((1, 1, 3 * 128), lambda i: (i, 0, 0)),
            blk((1, NWIN, 128), lambda i: (i, 0, 0)),
        ],
        out_specs=[
            blk((128, HIDDEN), lambda i: (i, 0)),
            full3((NWIN, HIDDEN)),
        ],
        out_shape=[
            jax.ShapeDtypeStruct((L, HIDDEN), jnp.float32),
            jax.ShapeDtypeStruct((NWIN, HIDDEN), jnp.float32),
        ],
        scratch_shapes=[pltpu.VMEM((NWIN, HIDDEN), jnp.float32)],
        interpret=interpret,
    )(xs2d, xs2d, xs2d, lwq3, lwk3, lwv3, lbq3, lbk3, lbv3, lwot3, lbo, lg, lb,
      _SEGQ, _SEGK, _SMATT)

    h2d = pl.pallas_call(
        _global_body,
        grid=(NBLK,),
        in_specs=[
            blk((128, HIDDEN), lambda i: (i, 0)),
            full3((NWIN, HIDDEN)),
            full3((NHEADS, DH, HIDDEN)), full3((NHEADS, DH, HIDDEN)),
            full3((NHEADS, DH, HIDDEN)),
            full3((NHEADS, 1, DH)), full3((NHEADS, 1, DH)),
            full3((NHEADS, 1, DH)),
            full3((NHEADS, DH, HIDDEN)),
            full3((1, HIDDEN)), full3((1, HIDDEN)), full3((1, HIDDEN)),
        ],
        out_specs=blk((128, HIDDEN), lambda i: (i, 0)),
        out_shape=jax.ShapeDtypeStruct((L, HIDDEN), jnp.float32),
        scratch_shapes=[pltpu.VMEM((1, HIDDEN), jnp.float32)],
        interpret=interpret,
    )(refined, r, gwq3, gwk3, gwv3, gbq3, gbk3, gbv3, gwot3, gbo, gg, gb)

    return h2d


def kernel(x, coords, weight_params, l_Wqkv, l_bqkv, l_Wo, l_bo, l_g, l_b,
           g_Wqkv, g_bqkv, g_Wo, g_bo, g_g, g_b):
    del coords, weight_params
    x2d = x.reshape(L, HIDDEN)
    xs2d = _sc_gather(x2d, _PERM)
    h2d = _tc_pipeline(xs2d, l_Wqkv, l_bqkv, l_Wo, l_bo, l_g, l_b,
                       g_Wqkv, g_bqkv, g_Wo, g_bo, g_g, g_b)
    return h2d.reshape(1, L, HIDDEN)


# speedup vs baseline: 1.2021x; 1.2021x over previous
import functools
import math

import jax
import jax.numpy as jnp
import numpy as np
from jax import lax
from jax.experimental import pallas as pl
from jax.experimental.pallas import tpu as pltpu
from jax.experimental.pallas import tpu_sc as plsc

HIDDEN = 384
NHEADS = 8
DH = HIDDEN // NHEADS
DP = 128
HP = NHEADS * DP
CLUSTER_SIZE = 64
L = 4096
NBLK = L // 128
SCALE = 1.0 / math.sqrt(DH)
EPS = 1e-5


def _static_plan():
    n_cluster = max(1, L // CLUSTER_SIZE)
    labels = np.random.RandomState(0).randint(0, n_cluster, size=L)
    index = np.argsort(labels, kind="stable")
    window_sizes = np.bincount(labels).tolist()
    new_sizes = []
    for size in window_sizes:
        if size >= CLUSTER_SIZE * 2:
            num_splits = max(1, size // CLUSTER_SIZE)
            q, r = divmod(size, num_splits)
            new_sizes.extend([q + 1 if i < r else q for i in range(num_splits)])
        else:
            new_sizes.append(size)
    sizes = [s for s in new_sizes if s > 0]
    return index.astype(np.int32), sizes


_PERM_NP, _SIZES = _static_plan()
NWIN = len(_SIZES)

_SEG_NP = np.repeat(np.arange(NWIN, dtype=np.int32), _SIZES)

_SEGQ_NP = _SEG_NP.reshape(NBLK, 128, 1)

_SEGK_NP = np.full((NBLK, 1, 3 * 128), -1, dtype=np.int32)
for _i in range(NBLK):
    _lo = (_i - 1) * 128
    _hi = (_i + 2) * 128
    _s = max(_lo, 0)
    _e = min(_hi, L)
    _SEGK_NP[_i, 0, _s - _lo:_e - _lo] = _SEG_NP[_s:_e]

_SMATT_NP = np.zeros((NBLK, NWIN, 128), dtype=np.float32)
for _i in range(NBLK):
    for _r in range(128):
        _w = _SEG_NP[_i * 128 + _r]
        _SMATT_NP[_i, _w, _r] = 1.0 / _SIZES[_w]


def _nt(a, b):
    return lax.dot_general(a, b, (((1,), (1,)), ((), ())),
                           preferred_element_type=jnp.float32)


def _bf(t):
    return t.astype(jnp.bfloat16)


def _sc_gather(x2d, idx):
    rows_per_w = L // 32
    mesh = plsc.VectorSubcoreMesh(core_axis_name="c", subcore_axis_name="s",
                                  num_cores=2, num_subcores=16)

    @functools.partial(
        pl.kernel,
        out_type=jax.ShapeDtypeStruct((L, HIDDEN), jnp.float32),
        mesh=mesh,
        scratch_types=[
            pltpu.VMEM((rows_per_w,), jnp.int32),
            pltpu.VMEM((rows_per_w, HIDDEN), jnp.float32),
            pltpu.SemaphoreType.DMA,
        ],
    )
    def body(x_hbm, idx_hbm, out_hbm, idx_v, rows_v, sem):
        wid = lax.axis_index("s") * 2 + lax.axis_index("c")
        base = wid * rows_per_w
        pltpu.sync_copy(idx_hbm.at[pl.ds(base, rows_per_w)], idx_v)
        pltpu.async_copy(x_hbm.at[idx_v], rows_v, sem).wait()
        pltpu.sync_copy(rows_v, out_hbm.at[pl.ds(base, rows_per_w)])

    return body(x2d, idx)


def _attend_padded(xq_bf, xkv_bf, wqp, wkp, wvp, bqp, bkp, bvp, wop, mask):
    q = _bf(_nt(xq_bf, wqp) + bqp)
    k = _bf(_nt(xkv_bf, wkp) + bkp)
    v = _bf(_nt(xkv_bf, wvp) + bvp)
    ohs = []
    for h in range(NHEADS):
        sl = slice(h * DP, (h + 1) * DP)
        s = _nt(q[:, sl], k[:, sl])
        if mask is not None:
            s = jnp.where(mask, s, -1e30)
        m = jnp.max(s, axis=1, keepdims=True)
        e = jnp.exp(s - m)
        p = _bf(e / jnp.sum(e, axis=1, keepdims=True))
        ohs.append(jnp.dot(p, v[:, sl], preferred_element_type=jnp.float32))
    attn = _bf(jnp.concatenate(ohs, axis=1))
    return _nt(attn, wop)


def _layernorm(x, g, b):
    mu = jnp.mean(x, axis=1, keepdims=True)
    xc = x - mu
    var = jnp.mean(xc * xc, axis=1, keepdims=True)
    return xc * lax.rsqrt(var + EPS) * g + b


def _local_body(xs_p, xs_c, xs_n, wqp, wkp, wvp, bqp, bkp, bvp, wop, bo,
                lg, lb, segq, segk, smatt, refined_ref, r_ref, acc_ref):
    i = pl.program_id(0)
    xq = xs_c[...]
    xq_bf = _bf(xq)
    xkv_bf = jnp.concatenate([_bf(xs_p[...]), xq_bf, _bf(xs_n[...])], axis=0)
    mask = segq[0] == segk[0]
    o = _attend_padded(xq_bf, xkv_bf, wqp[...], wkp[...], wvp[...], bqp[...],
                       bkp[...], bvp[...], wop[...], mask) + bo[...]
    refined = _layernorm(xq + o, lg[...], lb[...])
    refined_ref[...] = refined
    part = jnp.dot(smatt[0], refined, preferred_element_type=jnp.float32)

    @pl.when(i == 0)
    def _():
        acc_ref[...] = part

    @pl.when(i > 0)
    def _():
        acc_ref[...] = acc_ref[...] + part

    @pl.when(i == NBLK - 1)
    def _():
        r_ref[...] = acc_ref[...]


def _global_body(refined, r, wqp, wkp, wvp, bqp, bkp, bvp, wop, bo, gg, gb,
                 h_ref, vec_ref):
    i = pl.program_id(0)

    @pl.when(i == 0)
    def _():
        rr = r[...]
        rr_bf = _bf(rr)
        o = _attend_padded(rr_bf, rr_bf, wqp[...], wkp[...], wvp[...],
                           bqp[...], bkp[...], bvp[...], wop[...],
                           None) + bo[...]
        rp = _layernorm(rr + o, gg[...], gb[...])
        vec_ref[...] = jnp.mean(rp, axis=0, keepdims=True)

    h_ref[...] = refined[...] + vec_ref[...]


def _pad_params(Wqkv, bqkv, Wo):
    wq, wk, wv = jnp.split(Wqkv, 3, axis=0)

    def padw(w, scale=1.0):
        w3 = (w * scale).reshape(NHEADS, DH, HIDDEN)
        w3 = jnp.pad(w3, ((0, 0), (0, DP - DH), (0, 0)))
        return _bf(w3.reshape(HP, HIDDEN))

    bq, bk, bv = jnp.split(bqkv, 3)

    def padb(b, scale=1.0):
        b2 = (b * scale).reshape(NHEADS, DH)
        b2 = jnp.pad(b2, ((0, 0), (0, DP - DH)))
        return b2.reshape(1, HP).astype(jnp.float32)

    wqp = padw(wq, SCALE)
    wkp = padw(wk)
    wvp = padw(wv)
    bqp = padb(bq, SCALE)
    bkp = padb(bk)
    bvp = padb(bv)
    wot = Wo.T.reshape(NHEADS, DH, HIDDEN)
    wot = jnp.pad(wot, ((0, 0), (0, DP - DH), (0, 0))).reshape(HP, HIDDEN)
    wop = _bf(wot.T)
    return wqp, wkp, wvp, bqp, bkp, bvp, wop


def _tc_pipeline(xs2d, l_Wqkv, l_bqkv, l_Wo, l_bo, l_g, l_b,
                 g_Wqkv, g_bqkv, g_Wo, g_bo, g_g, g_b, interpret=False):
    lp = _pad_params(l_Wqkv, l_bqkv, l_Wo)
    gp = _pad_params(g_Wqkv, g_bqkv, g_Wo)
    lbo = l_bo.reshape(1, HIDDEN)
    lg = l_g.reshape(1, HIDDEN)
    lb = l_b.reshape(1, HIDDEN)
    gbo = g_bo.reshape(1, HIDDEN)
    gg = g_g.reshape(1, HIDDEN)
    gb = g_b.reshape(1, HIDDEN)

    full = lambda shp: pl.BlockSpec(shp, lambda i: (0,) * len(shp))
    blk = lambda shp, im: pl.BlockSpec(shp, im)
    wspecs = [full((HP, HIDDEN)), full((HP, HIDDEN)), full((HP, HIDDEN)),
              full((1, HP)), full((1, HP)), full((1, HP)),
              full((HIDDEN, HP))]

    refined, r = pl.pallas_call(
        _local_body,
        grid=(NBLK,),
        in_specs=[
            blk((128, HIDDEN), lambda i: (jnp.maximum(i - 1, 0), 0)),
            blk((128, HIDDEN), lambda i: (i, 0)),
            blk((128, HIDDEN), lambda i: (jnp.minimum(i + 1, NBLK - 1), 0)),
            *wspecs,
            full((1, HIDDEN)), full((1, HIDDEN)), full((1, HIDDEN)),
            blk((1, 128, 1), lambda i: (i, 0, 0)),
            blk((1, 1, 3 * 128), lambda i: (i, 0, 0)),
            blk((1, NWIN, 128), lambda i: (i, 0, 0)),
        ],
        out_specs=[
            blk((128, HIDDEN), lambda i: (i, 0)),
            full((NWIN, HIDDEN)),
        ],
        out_shape=[
            jax.ShapeDtypeStruct((L, HIDDEN), jnp.float32),
            jax.ShapeDtypeStruct((NWIN, HIDDEN), jnp.float32),
        ],
        scratch_shapes=[pltpu.VMEM((NWIN, HIDDEN), jnp.float32)],
        interpret=interpret,
    )(xs2d, xs2d, xs2d, *lp, lbo, lg, lb,
      jnp.asarray(_SEGQ_NP), jnp.asarray(_SEGK_NP), jnp.asarray(_SMATT_NP))

    h2d = pl.pallas_call(
        _global_body,
        grid=(NBLK,),
        in_specs=[
            blk((128, HIDDEN), lambda i: (i, 0)),
            full((NWIN, HIDDEN)),
            *wspecs,
            full((1, HIDDEN)), full((1, HIDDEN)), full((1, HIDDEN)),
        ],
        out_specs=blk((128, HIDDEN), lambda i: (i, 0)),
        out_shape=jax.ShapeDtypeStruct((L, HIDDEN), jnp.float32),
        scratch_shapes=[pltpu.VMEM((1, HIDDEN), jnp.float32)],
        interpret=interpret,
    )(refined, r, *gp, gbo, gg, gb)

    return h2d


def kernel(x, coords, weight_params, l_Wqkv, l_bqkv, l_Wo, l_bo, l_g, l_b,
           g_Wqkv, g_bqkv, g_Wo, g_bo, g_g, g_b):
    del coords, weight_params
    x2d = x.reshape(L, HIDDEN)
    xs2d = _sc_gather(x2d, jnp.asarray(_PERM_NP))
    h2d = _tc_pipeline(xs2d, l_Wqkv, l_bqkv, l_Wo, l_bo, l_g, l_b,
                       g_Wqkv, g_bqkv, g_Wo, g_bo, g_g, g_b)
    return h2d.reshape(1, L, HIDDEN)


# speedup vs baseline: 1.3510x; 1.1239x over previous
import functools
import math

import jax
import jax.numpy as jnp
import numpy as np
from jax import lax
from jax.experimental import pallas as pl
from jax.experimental.pallas import tpu as pltpu
from jax.experimental.pallas import tpu_sc as plsc

HIDDEN = 384
NHEADS = 8
DH = HIDDEN // NHEADS
DP = 128
HP = NHEADS * DP
CLUSTER_SIZE = 64
L = 4096
NBLK = L // 128
SCALE = 1.0 / math.sqrt(DH)
EPS = 1e-5


def _static_plan():
    n_cluster = max(1, L // CLUSTER_SIZE)
    labels = np.random.RandomState(0).randint(0, n_cluster, size=L)
    index = np.argsort(labels, kind="stable")
    window_sizes = np.bincount(labels).tolist()
    new_sizes = []
    for size in window_sizes:
        if size >= CLUSTER_SIZE * 2:
            num_splits = max(1, size // CLUSTER_SIZE)
            q, r = divmod(size, num_splits)
            new_sizes.extend([q + 1 if i < r else q for i in range(num_splits)])
        else:
            new_sizes.append(size)
    sizes = [s for s in new_sizes if s > 0]
    return index.astype(np.int32), sizes


_PERM_NP, _SIZES = _static_plan()
NWIN = len(_SIZES)

_SEG_NP = np.repeat(np.arange(NWIN, dtype=np.int32), _SIZES)

_SEGQ_NP = _SEG_NP.reshape(NBLK, 128, 1)

_SEGK_NP = np.full((NBLK, 1, 3 * 128), -1, dtype=np.int32)
for _i in range(NBLK):
    _lo = (_i - 1) * 128
    _hi = (_i + 2) * 128
    _s = max(_lo, 0)
    _e = min(_hi, L)
    _SEGK_NP[_i, 0, _s - _lo:_e - _lo] = _SEG_NP[_s:_e]

_SMATT_NP = np.zeros((NBLK, NWIN, 128), dtype=np.float32)
for _i in range(NBLK):
    for _r in range(128):
        _w = _SEG_NP[_i * 128 + _r]
        _SMATT_NP[_i, _w, _r] = 1.0 / _SIZES[_w]

_BIASROT_NP = np.full((NBLK, 128, 3 * 128), -1e30, dtype=np.float32)
for _i in range(NBLK):
    for _c in (_i - 1, _i, _i + 1):
        if 0 <= _c < NBLK:
            _j = _c % 3
            _mq = _SEG_NP[_i * 128:(_i + 1) * 128][:, None]
            _mk = _SEG_NP[_c * 128:(_c + 1) * 128][None, :]
            _BIASROT_NP[_i][:, _j * 128:(_j + 1) * 128] = np.where(
                _mq == _mk, 0.0, -1e30)


def _nt(a, b):
    return lax.dot_general(a, b, (((1,), (1,)), ((), ())),
                           preferred_element_type=jnp.float32)


def _bf(t):
    return t.astype(jnp.bfloat16)


def _sc_gather(x2d, idx):
    rows_per_w = L // 32
    mesh = plsc.VectorSubcoreMesh(core_axis_name="c", subcore_axis_name="s",
                                  num_cores=2, num_subcores=16)

    @functools.partial(
        pl.kernel,
        out_type=jax.ShapeDtypeStruct((L, HIDDEN), jnp.float32),
        mesh=mesh,
        scratch_types=[
            pltpu.VMEM((rows_per_w,), jnp.int32),
            pltpu.VMEM((rows_per_w, HIDDEN), jnp.float32),
            pltpu.SemaphoreType.DMA,
        ],
    )
    def body(x_hbm, idx_hbm, out_hbm, idx_v, rows_v, sem):
        wid = lax.axis_index("s") * 2 + lax.axis_index("c")
        base = wid * rows_per_w
        pltpu.sync_copy(idx_hbm.at[pl.ds(base, rows_per_w)], idx_v)
        pltpu.async_copy(x_hbm.at[idx_v], rows_v, sem).wait()
        pltpu.sync_copy(rows_v, out_hbm.at[pl.ds(base, rows_per_w)])

    return body(x2d, idx)


def _attend_padded(xq_bf, xkv_bf, wqp, wkp, wvp, bqp, bkp, bvp, wop, mask):
    q = _bf(_nt(xq_bf, wqp) + bqp)
    k = _bf(_nt(xkv_bf, wkp) + bkp)
    v = _bf(_nt(xkv_bf, wvp) + bvp)
    ohs = []
    for h in range(NHEADS):
        sl = slice(h * DP, (h + 1) * DP)
        s = _nt(q[:, sl], k[:, sl])
        if mask is not None:
            s = jnp.where(mask, s, -1e30)
        m = jnp.max(s, axis=1, keepdims=True)
        e = jnp.exp(s - m)
        p = _bf(e / jnp.sum(e, axis=1, keepdims=True))
        ohs.append(jnp.dot(p, v[:, sl], preferred_element_type=jnp.float32))
    attn = _bf(jnp.concatenate(ohs, axis=1))
    return _nt(attn, wop)


def _layernorm(x, g, b):
    mu = jnp.mean(x, axis=1, keepdims=True)
    xc = x - mu
    var = jnp.mean(xc * xc, axis=1, keepdims=True)
    return xc * lax.rsqrt(var + EPS) * g + b


def _local_body(xs_c, xs_n, wqp, wkp, wvp, bqp, bkp, bvp, wop, bo,
                lg, lb, biasrot, smatt, refined_ref, r_ref,
                qr_ref, kr_ref, vr_ref, acc_ref):
    i = pl.program_id(0)
    wq, wk, wv = wqp[...], wkp[...], wvp[...]
    bq, bk, bv = bqp[...], bkp[...], bvp[...]

    def qkv_into_slot(x_bf, slot):
        qr_ref[slot] = _bf(_nt(x_bf, wq) + bq)
        kr_ref[slot] = _bf(_nt(x_bf, wk) + bk)
        vr_ref[slot] = _bf(_nt(x_bf, wv) + bv)

    @pl.when(i == 0)
    def _():
        qkv_into_slot(_bf(xs_c[...]), 0)
        kr_ref[2] = jnp.zeros((128, HP), jnp.bfloat16)
        vr_ref[2] = jnp.zeros((128, HP), jnp.bfloat16)

    @pl.when(i < NBLK - 1)
    def _():
        qkv_into_slot(_bf(xs_n[...]), lax.rem(i + 1, 3))

    q_cur = qr_ref[lax.rem(i, 3)]
    kall = jnp.concatenate([kr_ref[0], kr_ref[1], kr_ref[2]], axis=0)
    vall = jnp.concatenate([vr_ref[0], vr_ref[1], vr_ref[2]], axis=0)
    bias = biasrot[0]

    ohs = []
    for h in range(NHEADS):
        sl = slice(h * DP, (h + 1) * DP)
        s = _nt(q_cur[:, sl], kall[:, sl]) + bias
        e = jnp.exp(s)
        p = _bf(e / jnp.sum(e, axis=1, keepdims=True))
        ohs.append(jnp.dot(p, vall[:, sl], preferred_element_type=jnp.float32))
    attn = _bf(jnp.concatenate(ohs, axis=1))
    o = _nt(attn, wop[...]) + bo[...]

    xq = xs_c[...]
    refined = _layernorm(xq + o, lg[...], lb[...])
    refined_ref[...] = refined
    part = jnp.dot(smatt[0], refined, preferred_element_type=jnp.float32)

    @pl.when(i == 0)
    def _():
        acc_ref[...] = part

    @pl.when(i > 0)
    def _():
        acc_ref[...] = acc_ref[...] + part

    @pl.when(i == NBLK - 1)
    def _():
        r_ref[...] = acc_ref[...]


def _global_body(refined, r, wqp, wkp, wvp, bqp, bkp, bvp, wop, bo, gg, gb,
                 h_ref, vec_ref):
    i = pl.program_id(0)

    @pl.when(i == 0)
    def _():
        rr = r[...]
        rr_bf = _bf(rr)
        o = _attend_padded(rr_bf, rr_bf, wqp[...], wkp[...], wvp[...],
                           bqp[...], bkp[...], bvp[...], wop[...],
                           None) + bo[...]
        rp = _layernorm(rr + o, gg[...], gb[...])
        vec_ref[...] = jnp.mean(rp, axis=0, keepdims=True)

    h_ref[...] = refined[...] + vec_ref[...]


def _pad_params(Wqkv, bqkv, Wo):
    wq, wk, wv = jnp.split(Wqkv, 3, axis=0)

    def padw(w, scale=1.0):
        w3 = (w * scale).reshape(NHEADS, DH, HIDDEN)
        w3 = jnp.pad(w3, ((0, 0), (0, DP - DH), (0, 0)))
        return _bf(w3.reshape(HP, HIDDEN))

    bq, bk, bv = jnp.split(bqkv, 3)

    def padb(b, scale=1.0):
        b2 = (b * scale).reshape(NHEADS, DH)
        b2 = jnp.pad(b2, ((0, 0), (0, DP - DH)))
        return b2.reshape(1, HP).astype(jnp.float32)

    wqp = padw(wq, SCALE)
    wkp = padw(wk)
    wvp = padw(wv)
    bqp = padb(bq, SCALE)
    bkp = padb(bk)
    bvp = padb(bv)
    wot = Wo.T.reshape(NHEADS, DH, HIDDEN)
    wot = jnp.pad(wot, ((0, 0), (0, DP - DH), (0, 0))).reshape(HP, HIDDEN)
    wop = _bf(wot.T)
    return wqp, wkp, wvp, bqp, bkp, bvp, wop


def _tc_pipeline(xs2d, l_Wqkv, l_bqkv, l_Wo, l_bo, l_g, l_b,
                 g_Wqkv, g_bqkv, g_Wo, g_bo, g_g, g_b, interpret=False):
    lp = _pad_params(l_Wqkv, l_bqkv, l_Wo)
    gp = _pad_params(g_Wqkv, g_bqkv, g_Wo)
    lbo = l_bo.reshape(1, HIDDEN)
    lg = l_g.reshape(1, HIDDEN)
    lb = l_b.reshape(1, HIDDEN)
    gbo = g_bo.reshape(1, HIDDEN)
    gg = g_g.reshape(1, HIDDEN)
    gb = g_b.reshape(1, HIDDEN)

    full = lambda shp: pl.BlockSpec(shp, lambda i: (0,) * len(shp))
    blk = lambda shp, im: pl.BlockSpec(shp, im)
    wspecs = [full((HP, HIDDEN)), full((HP, HIDDEN)), full((HP, HIDDEN)),
              full((1, HP)), full((1, HP)), full((1, HP)),
              full((HIDDEN, HP))]

    refined, r = pl.pallas_call(
        _local_body,
        grid=(NBLK,),
        in_specs=[
            blk((128, HIDDEN), lambda i: (i, 0)),
            blk((128, HIDDEN), lambda i: (jnp.minimum(i + 1, NBLK - 1), 0)),
            *wspecs,
            full((1, HIDDEN)), full((1, HIDDEN)), full((1, HIDDEN)),
            blk((1, 128, 3 * 128), lambda i: (i, 0, 0)),
            blk((1, NWIN, 128), lambda i: (i, 0, 0)),
        ],
        out_specs=[
            blk((128, HIDDEN), lambda i: (i, 0)),
            full((NWIN, HIDDEN)),
        ],
        out_shape=[
            jax.ShapeDtypeStruct((L, HIDDEN), jnp.float32),
            jax.ShapeDtypeStruct((NWIN, HIDDEN), jnp.float32),
        ],
        scratch_shapes=[
            pltpu.VMEM((3, 128, HP), jnp.bfloat16),
            pltpu.VMEM((3, 128, HP), jnp.bfloat16),
            pltpu.VMEM((3, 128, HP), jnp.bfloat16),
            pltpu.VMEM((NWIN, HIDDEN), jnp.float32),
        ],
        interpret=interpret,
    )(xs2d, xs2d, *lp, lbo, lg, lb,
      jnp.asarray(_BIASROT_NP), jnp.asarray(_SMATT_NP))

    h2d = pl.pallas_call(
        _global_body,
        grid=(NBLK,),
        in_specs=[
            blk((128, HIDDEN), lambda i: (i, 0)),
            full((NWIN, HIDDEN)),
            *wspecs,
            full((1, HIDDEN)), full((1, HIDDEN)), full((1, HIDDEN)),
        ],
        out_specs=blk((128, HIDDEN), lambda i: (i, 0)),
        out_shape=jax.ShapeDtypeStruct((L, HIDDEN), jnp.float32),
        scratch_shapes=[pltpu.VMEM((1, HIDDEN), jnp.float32)],
        interpret=interpret,
    )(refined, r, *gp, gbo, gg, gb)

    return h2d


def kernel(x, coords, weight_params, l_Wqkv, l_bqkv, l_Wo, l_bo, l_g, l_b,
           g_Wqkv, g_bqkv, g_Wo, g_bo, g_g, g_b):
    del coords, weight_params
    x2d = x.reshape(L, HIDDEN)
    xs2d = _sc_gather(x2d, jnp.asarray(_PERM_NP))
    h2d = _tc_pipeline(xs2d, l_Wqkv, l_bqkv, l_Wo, l_bo, l_g, l_b,
                       g_Wqkv, g_bqkv, g_Wo, g_bo, g_g, g_b)
    return h2d.reshape(1, L, HIDDEN)


# speedup vs baseline: 1.5385x; 1.1387x over previous
import functools
import math

import jax
import jax.numpy as jnp
import numpy as np
from jax import lax
from jax.experimental import pallas as pl
from jax.experimental.pallas import tpu as pltpu
from jax.experimental.pallas import tpu_sc as plsc

HIDDEN = 384
NHEADS = 8
DH = HIDDEN // NHEADS
DP = 128
HP = NHEADS * DP
CLUSTER_SIZE = 64
L = 4096
NBLK = L // 128
SCALE = 1.0 / math.sqrt(DH)
EPS = 1e-5


def _static_plan():
    n_cluster = max(1, L // CLUSTER_SIZE)
    labels = np.random.RandomState(0).randint(0, n_cluster, size=L)
    index = np.argsort(labels, kind="stable")
    window_sizes = np.bincount(labels).tolist()
    new_sizes = []
    for size in window_sizes:
        if size >= CLUSTER_SIZE * 2:
            num_splits = max(1, size // CLUSTER_SIZE)
            q, r = divmod(size, num_splits)
            new_sizes.extend([q + 1 if i < r else q for i in range(num_splits)])
        else:
            new_sizes.append(size)
    sizes = [s for s in new_sizes if s > 0]
    return index.astype(np.int32), sizes


_PERM_NP, _SIZES = _static_plan()
NWIN = len(_SIZES)

_SEG_NP = np.repeat(np.arange(NWIN, dtype=np.int32), _SIZES)

_SEGQ_NP = _SEG_NP.reshape(NBLK, 128, 1)

_SEGK_NP = np.full((NBLK, 1, 3 * 128), -1, dtype=np.int32)
for _i in range(NBLK):
    _lo = (_i - 1) * 128
    _hi = (_i + 2) * 128
    _s = max(_lo, 0)
    _e = min(_hi, L)
    _SEGK_NP[_i, 0, _s - _lo:_e - _lo] = _SEG_NP[_s:_e]

_SMATT_NP = np.zeros((NBLK, NWIN, 128), dtype=np.float32)
for _i in range(NBLK):
    for _r in range(128):
        _w = _SEG_NP[_i * 128 + _r]
        _SMATT_NP[_i, _w, _r] = 1.0 / _SIZES[_w]

_BIASROT_NP = np.full((NBLK, 128, 4 * 128), -1e30, dtype=np.float32)
for _b in range(NBLK):
    for _c in (_b - 1, _b, _b + 1):
        if 0 <= _c < NBLK:
            _j = _c % 4
            _mq = _SEG_NP[_b * 128:(_b + 1) * 128][:, None]
            _mk = _SEG_NP[_c * 128:(_c + 1) * 128][None, :]
            _BIASROT_NP[_b][:, _j * 128:(_j + 1) * 128] = np.where(
                _mq == _mk, 0.0, -1e30)


def _nt(a, b):
    return lax.dot_general(a, b, (((1,), (1,)), ((), ())),
                           preferred_element_type=jnp.float32)


def _bf(t):
    return t.astype(jnp.bfloat16)


def _sc_gather(x2d, idx):
    rows_per_w = L // 32
    mesh = plsc.VectorSubcoreMesh(core_axis_name="c", subcore_axis_name="s",
                                  num_cores=2, num_subcores=16)

    @functools.partial(
        pl.kernel,
        out_type=jax.ShapeDtypeStruct((L, HIDDEN), jnp.float32),
        mesh=mesh,
        scratch_types=[
            pltpu.VMEM((rows_per_w,), jnp.int32),
            pltpu.VMEM((rows_per_w, HIDDEN), jnp.float32),
            pltpu.SemaphoreType.DMA,
        ],
    )
    def body(x_hbm, idx_hbm, out_hbm, idx_v, rows_v, sem):
        wid = lax.axis_index("s") * 2 + lax.axis_index("c")
        base = wid * rows_per_w
        pltpu.sync_copy(idx_hbm.at[pl.ds(base, rows_per_w)], idx_v)
        pltpu.async_copy(x_hbm.at[idx_v], rows_v, sem).wait()
        pltpu.sync_copy(rows_v, out_hbm.at[pl.ds(base, rows_per_w)])

    return body(x2d, idx)


def _attend_padded(xq_bf, xkv_bf, wqp, wkp, wvp, bqp, bkp, bvp, wop, mask):
    q = _bf(_nt(xq_bf, wqp) + bqp)
    k = _bf(_nt(xkv_bf, wkp) + bkp)
    v = _bf(_nt(xkv_bf, wvp) + bvp)
    ones_bf = jnp.ones((xkv_bf.shape[0], DP), jnp.bfloat16)
    ohs = []
    for h in range(NHEADS):
        sl = slice(h * DP, (h + 1) * DP)
        s = _nt(q[:, sl], k[:, sl])
        if mask is not None:
            s = jnp.where(mask, s, -1e30)
        e = _bf(jnp.exp(s))
        oh = jnp.dot(e, v[:, sl], preferred_element_type=jnp.float32)
        den = jnp.dot(e, ones_bf, preferred_element_type=jnp.float32)
        ohs.append(_bf(oh / den))
    attn = jnp.concatenate(ohs, axis=1)
    return _nt(attn, wop)


def _layernorm(x, g, b):
    mu = jnp.mean(x, axis=1, keepdims=True)
    xc = x - mu
    var = jnp.mean(xc * xc, axis=1, keepdims=True)
    return xc * lax.rsqrt(var + EPS) * g + b


def _local_body(xs_a, xs_b, wqp, wkp, wvp, bqp, bkp, bvp, wop, bo,
                lg, lb, biasrot, smatt, refined_ref, r_ref,
                qr_ref, kr_ref, vr_ref, acc_ref):
    i = pl.program_id(0)
    b = i - 1

    q_cur = qr_ref[lax.rem(jnp.maximum(b, 0), 4)]
    kall = jnp.concatenate(
        [kr_ref[0], kr_ref[1], kr_ref[2], kr_ref[3]], axis=0)
    vall = jnp.concatenate(
        [vr_ref[0], vr_ref[1], vr_ref[2], vr_ref[3]], axis=0)
    bias = biasrot[0]
    ones_bf = jnp.ones((4 * 128, DP), jnp.bfloat16)
    ohs = []
    for h in range(NHEADS):
        sl = slice(h * DP, (h + 1) * DP)
        s = _nt(q_cur[:, sl], kall[:, sl]) + bias
        e = _bf(jnp.exp(s))
        oh = jnp.dot(e, vall[:, sl], preferred_element_type=jnp.float32)
        den = jnp.dot(e, ones_bf, preferred_element_type=jnp.float32)
        ohs.append(_bf(oh / den))
    attn = jnp.concatenate(ohs, axis=1)
    o = _nt(attn, wop[...]) + bo[...]
    refined = _layernorm(xs_a[...] + o, lg[...], lb[...])
    refined_ref[...] = refined
    part = jnp.dot(smatt[0], refined, preferred_element_type=jnp.float32)

    wq, wk, wv = wqp[...], wkp[...], wvp[...]
    bq, bk, bv = bqp[...], bkp[...], bvp[...]

    def qkv_into_slot(x_bf, slot):
        qr_ref[slot] = _bf(_nt(x_bf, wq) + bq)
        kr_ref[slot] = _bf(_nt(x_bf, wk) + bk)
        vr_ref[slot] = _bf(_nt(x_bf, wv) + bv)

    qkv_into_slot(_bf(xs_b[...]), lax.rem(i + 1, 4))

    @pl.when(b == 0)
    def _():
        acc_ref[...] = part

    @pl.when(b > 0)
    def _():
        acc_ref[...] = acc_ref[...] + part

    @pl.when(b == NBLK - 1)
    def _():
        r_ref[...] = acc_ref[...]

    @pl.when(i == 0)
    def _():
        qkv_into_slot(_bf(xs_a[...]), 0)
        kr_ref[2] = jnp.zeros((128, HP), jnp.bfloat16)
        vr_ref[2] = jnp.zeros((128, HP), jnp.bfloat16)
        kr_ref[3] = jnp.zeros((128, HP), jnp.bfloat16)
        vr_ref[3] = jnp.zeros((128, HP), jnp.bfloat16)


def _global_body(refined, r, wqp, wkp, wvp, bqp, bkp, bvp, wop, bo, gg, gb,
                 h_ref, vec_ref):
    i = pl.program_id(0)

    @pl.when(i == 0)
    def _():
        rr = r[...]
        rr_bf = _bf(rr)
        o = _attend_padded(rr_bf, rr_bf, wqp[...], wkp[...], wvp[...],
                           bqp[...], bkp[...], bvp[...], wop[...],
                           None) + bo[...]
        rp = _layernorm(rr + o, gg[...], gb[...])
        vec_ref[...] = jnp.mean(rp, axis=0, keepdims=True)

    h_ref[...] = refined[...] + vec_ref[...]


def _pad_params(Wqkv, bqkv, Wo):
    wq, wk, wv = jnp.split(Wqkv, 3, axis=0)

    def padw(w, scale=1.0):
        w3 = (w * scale).reshape(NHEADS, DH, HIDDEN)
        w3 = jnp.pad(w3, ((0, 0), (0, DP - DH), (0, 0)))
        return _bf(w3.reshape(HP, HIDDEN))

    bq, bk, bv = jnp.split(bqkv, 3)

    def padb(b, scale=1.0):
        b2 = (b * scale).reshape(NHEADS, DH)
        b2 = jnp.pad(b2, ((0, 0), (0, DP - DH)))
        return b2.reshape(1, HP).astype(jnp.float32)

    wqp = padw(wq, SCALE)
    wkp = padw(wk)
    wvp = padw(wv)
    bqp = padb(bq, SCALE)
    bkp = padb(bk)
    bvp = padb(bv)
    wot = Wo.T.reshape(NHEADS, DH, HIDDEN)
    wot = jnp.pad(wot, ((0, 0), (0, DP - DH), (0, 0))).reshape(HP, HIDDEN)
    wop = _bf(wot.T)
    return wqp, wkp, wvp, bqp, bkp, bvp, wop


def _tc_pipeline(xs2d, l_Wqkv, l_bqkv, l_Wo, l_bo, l_g, l_b,
                 g_Wqkv, g_bqkv, g_Wo, g_bo, g_g, g_b, interpret=False):
    lp = _pad_params(l_Wqkv, l_bqkv, l_Wo)
    gp = _pad_params(g_Wqkv, g_bqkv, g_Wo)
    lbo = l_bo.reshape(1, HIDDEN)
    lg = l_g.reshape(1, HIDDEN)
    lb = l_b.reshape(1, HIDDEN)
    gbo = g_bo.reshape(1, HIDDEN)
    gg = g_g.reshape(1, HIDDEN)
    gb = g_b.reshape(1, HIDDEN)

    full = lambda shp: pl.BlockSpec(shp, lambda i: (0,) * len(shp))
    blk = lambda shp, im: pl.BlockSpec(shp, im)
    wspecs = [full((HP, HIDDEN)), full((HP, HIDDEN)), full((HP, HIDDEN)),
              full((1, HP)), full((1, HP)), full((1, HP)),
              full((HIDDEN, HP))]

    refined, r = pl.pallas_call(
        _local_body,
        grid=(NBLK + 1,),
        in_specs=[
            blk((128, HIDDEN), lambda i: (jnp.maximum(i - 1, 0), 0)),
            blk((128, HIDDEN), lambda i: (jnp.minimum(i + 1, NBLK - 1), 0)),
            *wspecs,
            full((1, HIDDEN)), full((1, HIDDEN)), full((1, HIDDEN)),
            blk((1, 128, 4 * 128), lambda i: (jnp.maximum(i - 1, 0), 0, 0)),
            blk((1, NWIN, 128), lambda i: (jnp.maximum(i - 1, 0), 0, 0)),
        ],
        out_specs=[
            blk((128, HIDDEN), lambda i: (jnp.maximum(i - 1, 0), 0)),
            full((NWIN, HIDDEN)),
        ],
        out_shape=[
            jax.ShapeDtypeStruct((L, HIDDEN), jnp.float32),
            jax.ShapeDtypeStruct((NWIN, HIDDEN), jnp.float32),
        ],
        scratch_shapes=[
            pltpu.VMEM((4, 128, HP), jnp.bfloat16),
            pltpu.VMEM((4, 128, HP), jnp.bfloat16),
            pltpu.VMEM((4, 128, HP), jnp.bfloat16),
            pltpu.VMEM((NWIN, HIDDEN), jnp.float32),
        ],
        interpret=interpret,
    )(xs2d, xs2d, *lp, lbo, lg, lb,
      jnp.asarray(_BIASROT_NP), jnp.asarray(_SMATT_NP))

    h2d = pl.pallas_call(
        _global_body,
        grid=(NBLK,),
        in_specs=[
            blk((128, HIDDEN), lambda i: (i, 0)),
            full((NWIN, HIDDEN)),
            *wspecs,
            full((1, HIDDEN)), full((1, HIDDEN)), full((1, HIDDEN)),
        ],
        out_specs=blk((128, HIDDEN), lambda i: (i, 0)),
        out_shape=jax.ShapeDtypeStruct((L, HIDDEN), jnp.float32),
        scratch_shapes=[pltpu.VMEM((1, HIDDEN), jnp.float32)],
        interpret=interpret,
    )(refined, r, *gp, gbo, gg, gb)

    return h2d


def kernel(x, coords, weight_params, l_Wqkv, l_bqkv, l_Wo, l_bo, l_g, l_b,
           g_Wqkv, g_bqkv, g_Wo, g_bo, g_g, g_b):
    del coords, weight_params
    x2d = x.reshape(L, HIDDEN)
    xs2d = _sc_gather(x2d, jnp.asarray(_PERM_NP))
    h2d = _tc_pipeline(xs2d, l_Wqkv, l_bqkv, l_Wo, l_bo, l_g, l_b,
                       g_Wqkv, g_bqkv, g_Wo, g_bo, g_g, g_b)
    return h2d.reshape(1, L, HIDDEN)


# speedup vs baseline: 1.5929x; 1.0354x over previous
import functools
import math

import jax
import jax.numpy as jnp
import numpy as np
from jax import lax
from jax.experimental import pallas as pl
from jax.experimental.pallas import tpu as pltpu
from jax.experimental.pallas import tpu_sc as plsc

HIDDEN = 384
NHEADS = 8
DH = HIDDEN // NHEADS
DP = 128
HP = NHEADS * DP
CLUSTER_SIZE = 64
L = 4096
NBLK = L // 128
SCALE = 1.0 / math.sqrt(DH)
EPS = 1e-5


def _static_plan():
    n_cluster = max(1, L // CLUSTER_SIZE)
    labels = np.random.RandomState(0).randint(0, n_cluster, size=L)
    index = np.argsort(labels, kind="stable")
    window_sizes = np.bincount(labels).tolist()
    new_sizes = []
    for size in window_sizes:
        if size >= CLUSTER_SIZE * 2:
            num_splits = max(1, size // CLUSTER_SIZE)
            q, r = divmod(size, num_splits)
            new_sizes.extend([q + 1 if i < r else q for i in range(num_splits)])
        else:
            new_sizes.append(size)
    sizes = [s for s in new_sizes if s > 0]
    return index.astype(np.int32), sizes


_PERM_NP, _SIZES = _static_plan()
NWIN = len(_SIZES)

_SEG_NP = np.repeat(np.arange(NWIN, dtype=np.int32), _SIZES)

_SEGQ_NP = _SEG_NP.reshape(NBLK, 128, 1)

_SEGK_NP = np.full((NBLK, 1, 3 * 128), -1, dtype=np.int32)
for _i in range(NBLK):
    _lo = (_i - 1) * 128
    _hi = (_i + 2) * 128
    _s = max(_lo, 0)
    _e = min(_hi, L)
    _SEGK_NP[_i, 0, _s - _lo:_e - _lo] = _SEG_NP[_s:_e]

_SMATT_NP = np.zeros((NBLK, NWIN, 128), dtype=np.float32)
for _i in range(NBLK):
    for _r in range(128):
        _w = _SEG_NP[_i * 128 + _r]
        _SMATT_NP[_i, _w, _r] = 1.0 / _SIZES[_w]

_BIASROT_NP = np.full((NBLK, 128, 4 * 128), -1e30, dtype=np.float32)
for _b in range(NBLK):
    for _c in (_b - 1, _b, _b + 1):
        if 0 <= _c < NBLK:
            _j = _c % 4
            _mq = _SEG_NP[_b * 128:(_b + 1) * 128][:, None]
            _mk = _SEG_NP[_c * 128:(_c + 1) * 128][None, :]
            _BIASROT_NP[_b][:, _j * 128:(_j + 1) * 128] = np.where(
                _mq == _mk, 0.0, -1e30)


def _nt(a, b):
    return lax.dot_general(a, b, (((1,), (1,)), ((), ())),
                           preferred_element_type=jnp.float32)


def _bf(t):
    return t.astype(jnp.bfloat16)


def _sc_gather(x2d, idx):
    rows_per_w = L // 32
    mesh = plsc.VectorSubcoreMesh(core_axis_name="c", subcore_axis_name="s",
                                  num_cores=2, num_subcores=16)

    @functools.partial(
        pl.kernel,
        out_type=jax.ShapeDtypeStruct((L, HIDDEN), jnp.float32),
        mesh=mesh,
        scratch_types=[
            pltpu.VMEM((rows_per_w,), jnp.int32),
            pltpu.VMEM((rows_per_w, HIDDEN), jnp.float32),
            pltpu.SemaphoreType.DMA,
        ],
    )
    def body(x_hbm, idx_hbm, out_hbm, idx_v, rows_v, sem):
        wid = lax.axis_index("s") * 2 + lax.axis_index("c")
        base = wid * rows_per_w
        pltpu.sync_copy(idx_hbm.at[pl.ds(base, rows_per_w)], idx_v)
        pltpu.async_copy(x_hbm.at[idx_v], rows_v, sem).wait()
        pltpu.sync_copy(rows_v, out_hbm.at[pl.ds(base, rows_per_w)])

    return body(x2d, idx)


def _attend_padded(xq_bf, xkv_bf, wqp, wkp, wvp, bqp, bkp, bvp, wop, mask):
    q = _bf(_nt(xq_bf, wqp) + bqp)
    k = _bf(_nt(xkv_bf, wkp) + bkp)
    v = _bf(_nt(xkv_bf, wvp) + bvp)
    ones_bf = jnp.ones((xkv_bf.shape[0], DP), jnp.bfloat16)
    ohs = []
    for h in range(NHEADS):
        sl = slice(h * DP, (h + 1) * DP)
        s = _nt(q[:, sl], k[:, sl])
        if mask is not None:
            s = jnp.where(mask, s, -1e30)
        e = _bf(jnp.exp(s))
        oh = jnp.dot(e, v[:, sl], preferred_element_type=jnp.float32)
        den = jnp.dot(e, ones_bf, preferred_element_type=jnp.float32)
        ohs.append(_bf(oh / den))
    attn = jnp.concatenate(ohs, axis=1)
    return _nt(attn, wop)


def _layernorm(x, g, b):
    mu = jnp.mean(x, axis=1, keepdims=True)
    xc = x - mu
    var = jnp.mean(xc * xc, axis=1, keepdims=True)
    return xc * lax.rsqrt(var + EPS) * g + b


def _local_body(xs_a, xs_b, wqkvp, bqkvp, wop, bo, lg, lb,
                gwqkvp, gbqkvp, gwop, gbo, gg, gb, biasrot, smatt,
                refined_ref, vec_ref, qr_ref, kr_ref, vr_ref, acc_ref):
    i = pl.program_id(0)
    b = i - 1

    q_cur = qr_ref[lax.rem(jnp.maximum(b, 0), 4)]
    kall = jnp.concatenate(
        [kr_ref[0], kr_ref[1], kr_ref[2], kr_ref[3]], axis=0)
    vall = jnp.concatenate(
        [vr_ref[0], vr_ref[1], vr_ref[2], vr_ref[3]], axis=0)
    bias = biasrot[0]
    ones_bf = jnp.ones((4 * 128, DP), jnp.bfloat16)
    ohs = []
    for h in range(NHEADS):
        sl = slice(h * DP, (h + 1) * DP)
        s = _nt(q_cur[:, sl], kall[:, sl]) + bias
        e = _bf(jnp.exp(s))
        oh = jnp.dot(e, vall[:, sl], preferred_element_type=jnp.float32)
        den = jnp.dot(e, ones_bf, preferred_element_type=jnp.float32)
        ohs.append(_bf(oh / den))
    attn = jnp.concatenate(ohs, axis=1)
    o = _nt(attn, wop[...]) + bo[...]
    refined = _layernorm(xs_a[...] + o, lg[...], lb[...])
    refined_ref[...] = refined
    part = jnp.dot(smatt[0], refined, preferred_element_type=jnp.float32)

    wqkv = wqkvp[...]
    bqkv = bqkvp[...]

    def qkv_into_slot(x_bf, slot):
        qkv = _nt(x_bf, wqkv) + bqkv
        qr_ref[slot] = _bf(qkv[:, :HP])
        kr_ref[slot] = _bf(qkv[:, HP:2 * HP])
        vr_ref[slot] = _bf(qkv[:, 2 * HP:])

    qkv_into_slot(_bf(xs_b[...]), lax.rem(i + 1, 4))

    @pl.when(b == 0)
    def _():
        acc_ref[...] = part

    @pl.when(b > 0)
    def _():
        acc_ref[...] = acc_ref[...] + part

    @pl.when(b == NBLK - 1)
    def _():
        rr = acc_ref[...]
        rr_bf = _bf(rr)
        gqkv = _nt(rr_bf, gwqkvp[...]) + gbqkvp[...]
        gq = _bf(gqkv[:, :HP])
        gk = _bf(gqkv[:, HP:2 * HP])
        gv = _bf(gqkv[:, 2 * HP:])
        gones = jnp.ones((NWIN, DP), jnp.bfloat16)
        gohs = []
        for h in range(NHEADS):
            sl = slice(h * DP, (h + 1) * DP)
            s = _nt(gq[:, sl], gk[:, sl])
            e = _bf(jnp.exp(s))
            oh = jnp.dot(e, gv[:, sl], preferred_element_type=jnp.float32)
            den = jnp.dot(e, gones, preferred_element_type=jnp.float32)
            gohs.append(_bf(oh / den))
        gattn = jnp.concatenate(gohs, axis=1)
        go = _nt(gattn, gwop[...]) + gbo[...]
        rp = _layernorm(rr + go, gg[...], gb[...])
        vec = jnp.mean(rp, axis=0, keepdims=True)
        vec_ref[...] = jnp.broadcast_to(vec, (8, HIDDEN))

    @pl.when(i == 0)
    def _():
        qkv_into_slot(_bf(xs_a[...]), 0)
        kr_ref[2] = jnp.zeros((128, HP), jnp.bfloat16)
        vr_ref[2] = jnp.zeros((128, HP), jnp.bfloat16)
        kr_ref[3] = jnp.zeros((128, HP), jnp.bfloat16)
        vr_ref[3] = jnp.zeros((128, HP), jnp.bfloat16)


def _add_body(refined, vec, h_ref):
    h_ref[...] = refined[...] + vec[0:1, :]


def _pad_params(Wqkv, bqkv, Wo):
    wq, wk, wv = jnp.split(Wqkv, 3, axis=0)

    def padw(w, scale=1.0):
        w3 = (w * scale).reshape(NHEADS, DH, HIDDEN)
        w3 = jnp.pad(w3, ((0, 0), (0, DP - DH), (0, 0)))
        return _bf(w3.reshape(HP, HIDDEN))

    bq, bk, bv = jnp.split(bqkv, 3)

    def padb(b, scale=1.0):
        b2 = (b * scale).reshape(NHEADS, DH)
        b2 = jnp.pad(b2, ((0, 0), (0, DP - DH)))
        return b2.reshape(1, HP).astype(jnp.float32)

    wqkvp = jnp.concatenate([padw(wq, SCALE), padw(wk), padw(wv)], axis=0)
    bqkvp = jnp.concatenate([padb(bq, SCALE), padb(bk), padb(bv)], axis=1)
    wot = Wo.T.reshape(NHEADS, DH, HIDDEN)
    wot = jnp.pad(wot, ((0, 0), (0, DP - DH), (0, 0))).reshape(HP, HIDDEN)
    wop = _bf(wot.T)
    return wqkvp, bqkvp, wop


def _tc_pipeline(xs2d, l_Wqkv, l_bqkv, l_Wo, l_bo, l_g, l_b,
                 g_Wqkv, g_bqkv, g_Wo, g_bo, g_g, g_b, interpret=False):
    lwqkvp, lbqkvp, lwop = _pad_params(l_Wqkv, l_bqkv, l_Wo)
    gwqkvp, gbqkvp, gwop = _pad_params(g_Wqkv, g_bqkv, g_Wo)
    lbo = l_bo.reshape(1, HIDDEN)
    lg = l_g.reshape(1, HIDDEN)
    lb = l_b.reshape(1, HIDDEN)
    gbo = g_bo.reshape(1, HIDDEN)
    gg = g_g.reshape(1, HIDDEN)
    gb = g_b.reshape(1, HIDDEN)

    full = lambda shp: pl.BlockSpec(shp, lambda i: (0,) * len(shp))
    blk = lambda shp, im: pl.BlockSpec(shp, im)

    refined, vec = pl.pallas_call(
        _local_body,
        grid=(NBLK + 1,),
        in_specs=[
            blk((128, HIDDEN), lambda i: (jnp.maximum(i - 1, 0), 0)),
            blk((128, HIDDEN), lambda i: (jnp.minimum(i + 1, NBLK - 1), 0)),
            full((3 * HP, HIDDEN)), full((1, 3 * HP)), full((HIDDEN, HP)),
            full((1, HIDDEN)), full((1, HIDDEN)), full((1, HIDDEN)),
            full((3 * HP, HIDDEN)), full((1, 3 * HP)), full((HIDDEN, HP)),
            full((1, HIDDEN)), full((1, HIDDEN)), full((1, HIDDEN)),
            blk((1, 128, 4 * 128), lambda i: (jnp.maximum(i - 1, 0), 0, 0)),
            blk((1, NWIN, 128), lambda i: (jnp.maximum(i - 1, 0), 0, 0)),
        ],
        out_specs=[
            blk((128, HIDDEN), lambda i: (jnp.maximum(i - 1, 0), 0)),
            full((8, HIDDEN)),
        ],
        out_shape=[
            jax.ShapeDtypeStruct((L, HIDDEN), jnp.float32),
            jax.ShapeDtypeStruct((8, HIDDEN), jnp.float32),
        ],
        scratch_shapes=[
            pltpu.VMEM((4, 128, HP), jnp.bfloat16),
            pltpu.VMEM((4, 128, HP), jnp.bfloat16),
            pltpu.VMEM((4, 128, HP), jnp.bfloat16),
            pltpu.VMEM((NWIN, HIDDEN), jnp.float32),
        ],
        interpret=interpret,
    )(xs2d, xs2d, lwqkvp, lbqkvp, lwop, lbo, lg, lb,
      gwqkvp, gbqkvp, gwop, gbo, gg, gb,
      jnp.asarray(_BIASROT_NP), jnp.asarray(_SMATT_NP))

    ADD_ROWS = 512
    h2d = pl.pallas_call(
        _add_body,
        grid=(L // ADD_ROWS,),
        in_specs=[
            blk((ADD_ROWS, HIDDEN), lambda i: (i, 0)),
            full((8, HIDDEN)),
        ],
        out_specs=blk((ADD_ROWS, HIDDEN), lambda i: (i, 0)),
        out_shape=jax.ShapeDtypeStruct((L, HIDDEN), jnp.float32),
        interpret=interpret,
    )(refined, vec)

    return h2d


def kernel(x, coords, weight_params, l_Wqkv, l_bqkv, l_Wo, l_bo, l_g, l_b,
           g_Wqkv, g_bqkv, g_Wo, g_bo, g_g, g_b):
    del coords, weight_params
    x2d = x.reshape(L, HIDDEN)
    xs2d = _sc_gather(x2d, jnp.asarray(_PERM_NP))
    h2d = _tc_pipeline(xs2d, l_Wqkv, l_bqkv, l_Wo, l_bo, l_g, l_b,
                       g_Wqkv, g_bqkv, g_Wo, g_bo, g_g, g_b)
    return h2d.reshape(1, L, HIDDEN)


# speedup vs baseline: 2.0879x; 1.3107x over previous
import functools
import math

import jax
import jax.numpy as jnp
import numpy as np
from jax import lax
from jax.experimental import pallas as pl
from jax.experimental.pallas import tpu as pltpu
from jax.experimental.pallas import tpu_sc as plsc

HIDDEN = 384
NHEADS = 8
DH = HIDDEN // NHEADS
DP = 128
HP = NHEADS * DP
CLUSTER_SIZE = 64
L = 4096
NBLK = L // 128
SCALE = 1.0 / math.sqrt(DH)
EPS = 1e-5


def _static_plan():
    n_cluster = max(1, L // CLUSTER_SIZE)
    labels = np.random.RandomState(0).randint(0, n_cluster, size=L)
    index = np.argsort(labels, kind="stable")
    window_sizes = np.bincount(labels).tolist()
    new_sizes = []
    for size in window_sizes:
        if size >= CLUSTER_SIZE * 2:
            num_splits = max(1, size // CLUSTER_SIZE)
            q, r = divmod(size, num_splits)
            new_sizes.extend([q + 1 if i < r else q for i in range(num_splits)])
        else:
            new_sizes.append(size)
    sizes = [s for s in new_sizes if s > 0]
    return index.astype(np.int32), sizes


_PERM_NP, _SIZES = _static_plan()
NWIN = len(_SIZES)

_SEG_NP = np.repeat(np.arange(NWIN, dtype=np.int32), _SIZES)

_SEGQ_NP = _SEG_NP.reshape(NBLK, 128, 1)

_SEGK_NP = np.full((NBLK, 1, 3 * 128), -1, dtype=np.int32)
for _i in range(NBLK):
    _lo = (_i - 1) * 128
    _hi = (_i + 2) * 128
    _s = max(_lo, 0)
    _e = min(_hi, L)
    _SEGK_NP[_i, 0, _s - _lo:_e - _lo] = _SEG_NP[_s:_e]

_SMATT_NP = np.zeros((NBLK, NWIN, 128), dtype=np.float32)
for _i in range(NBLK):
    for _r in range(128):
        _w = _SEG_NP[_i * 128 + _r]
        _SMATT_NP[_i, _w, _r] = 1.0 / _SIZES[_w]

_BIASROT_NP = np.full((NBLK // 2, 256, 4 * 128), -1e30, dtype=np.float32)
for _B in range(NBLK // 2):
    _mq = _SEG_NP[_B * 256:(_B + 1) * 256][:, None]
    for _t, _c in enumerate(range(2 * _B - 1, 2 * _B + 3)):
        if 0 <= _c < NBLK:
            _mk = _SEG_NP[_c * 128:(_c + 1) * 128][None, :]
            _BIASROT_NP[_B][:, _t * 128:(_t + 1) * 128] = np.where(
                _mq == _mk, 0.0, -1e30)

_SMATT2_NP = np.zeros((NBLK // 2, NWIN, 256), dtype=np.float32)
for _B in range(NBLK // 2):
    for _r in range(256):
        _w = _SEG_NP[_B * 256 + _r]
        _SMATT2_NP[_B, _w, _r] = 1.0 / _SIZES[_w]


def _nt(a, b):
    return lax.dot_general(a, b, (((1,), (1,)), ((), ())),
                           preferred_element_type=jnp.float32)


def _bf(t):
    return t.astype(jnp.bfloat16)


def _sc_gather(x2d, idx):
    rows_per_w = L // 32
    mesh = plsc.VectorSubcoreMesh(core_axis_name="c", subcore_axis_name="s",
                                  num_cores=2, num_subcores=16)

    @functools.partial(
        pl.kernel,
        out_type=jax.ShapeDtypeStruct((L, HIDDEN), jnp.float32),
        mesh=mesh,
        scratch_types=[
            pltpu.VMEM((rows_per_w,), jnp.int32),
            pltpu.VMEM((rows_per_w, HIDDEN), jnp.float32),
            pltpu.SemaphoreType.DMA,
        ],
    )
    def body(x_hbm, idx_hbm, out_hbm, idx_v, rows_v, sem):
        wid = lax.axis_index("s") * 2 + lax.axis_index("c")
        base = wid * rows_per_w
        pltpu.sync_copy(idx_hbm.at[pl.ds(base, rows_per_w)], idx_v)
        pltpu.async_copy(x_hbm.at[idx_v], rows_v, sem).wait()
        pltpu.sync_copy(rows_v, out_hbm.at[pl.ds(base, rows_per_w)])

    return body(x2d, idx)


def _attend_padded(xq_bf, xkv_bf, wqp, wkp, wvp, bqp, bkp, bvp, wop, mask):
    q = _bf(_nt(xq_bf, wqp) + bqp)
    k = _bf(_nt(xkv_bf, wkp) + bkp)
    v = _bf(_nt(xkv_bf, wvp) + bvp)
    ones_bf = jnp.ones((xkv_bf.shape[0], DP), jnp.bfloat16)
    ohs = []
    for h in range(NHEADS):
        sl = slice(h * DP, (h + 1) * DP)
        s = _nt(q[:, sl], k[:, sl])
        if mask is not None:
            s = jnp.where(mask, s, -1e30)
        e = _bf(jnp.exp(s))
        oh = jnp.dot(e, v[:, sl], preferred_element_type=jnp.float32)
        den = jnp.dot(e, ones_bf, preferred_element_type=jnp.float32)
        ohs.append(_bf(oh / den))
    attn = jnp.concatenate(ohs, axis=1)
    return _nt(attn, wop)


def _layernorm(x, g, b):
    mu = jnp.mean(x, axis=1, keepdims=True)
    xc = x - mu
    var = jnp.mean(xc * xc, axis=1, keepdims=True)
    return xc * lax.rsqrt(var + EPS) * g + b


def _local_body(xs_a, xs_b, wqkvp, bqkvp, wop, bo, lg, lb,
                gwqkvp, gbqkvp, gwop, gbo, gg, gb, biasrot, smatt,
                refined_ref, vec_ref, qr_ref, kr_ref, vr_ref, acc_ref):
    g = pl.program_id(0)
    B = jnp.maximum(g - 1, 0)

    q_cur = jnp.concatenate(
        [qr_ref[lax.rem(2 * B, 8)], qr_ref[lax.rem(2 * B + 1, 8)]], axis=0)
    kslots = [lax.rem(2 * B - 1 + t + 8, 8) for t in range(4)]
    kall = jnp.concatenate([kr_ref[s] for s in kslots], axis=0)
    vall = jnp.concatenate([vr_ref[s] for s in kslots], axis=0)
    bias = biasrot[0]
    ones_bf = jnp.ones((4 * 128, DP), jnp.bfloat16)
    ohs = []
    for h in range(NHEADS):
        sl = slice(h * DP, (h + 1) * DP)
        s = _nt(q_cur[:, sl], kall[:, sl]) + bias
        e = _bf(jnp.exp(s))
        oh = jnp.dot(e, vall[:, sl], preferred_element_type=jnp.float32)
        den = jnp.dot(e, ones_bf, preferred_element_type=jnp.float32)
        ohs.append(_bf(oh / den))
    attn = jnp.concatenate(ohs, axis=1)
    o = _nt(attn, wop[...]) + bo[...]
    refined = _layernorm(xs_a[...] + o, lg[...], lb[...])
    refined_ref[...] = refined
    part = jnp.dot(smatt[0], refined, preferred_element_type=jnp.float32)

    wqkv = wqkvp[...]
    bqkv = bqkvp[...]

    def qkv_into_slots(x_bf, s0, s1):
        qkv = _nt(x_bf, wqkv) + bqkv
        qr_ref[s0] = _bf(qkv[:128, :HP])
        qr_ref[s1] = _bf(qkv[128:, :HP])
        kr_ref[s0] = _bf(qkv[:128, HP:2 * HP])
        kr_ref[s1] = _bf(qkv[128:, HP:2 * HP])
        vr_ref[s0] = _bf(qkv[:128, 2 * HP:])
        vr_ref[s1] = _bf(qkv[128:, 2 * HP:])

    qkv_into_slots(_bf(xs_b[...]), lax.rem(2 * g + 2, 8),
                   lax.rem(2 * g + 3, 8))

    b2 = g - 1

    @pl.when(b2 == 0)
    def _():
        acc_ref[...] = part

    @pl.when(b2 > 0)
    def _():
        acc_ref[...] = acc_ref[...] + part

    @pl.when(b2 == NBLK // 2 - 1)
    def _():
        rr = acc_ref[...]
        rr_bf = _bf(rr)
        gqkv = _nt(rr_bf, gwqkvp[...]) + gbqkvp[...]
        gq = _bf(gqkv[:, :HP])
        gk = _bf(gqkv[:, HP:2 * HP])
        gv = _bf(gqkv[:, 2 * HP:])
        gones = jnp.ones((NWIN, DP), jnp.bfloat16)
        gohs = []
        for h in range(NHEADS):
            sl = slice(h * DP, (h + 1) * DP)
            s = _nt(gq[:, sl], gk[:, sl])
            e = _bf(jnp.exp(s))
            oh = jnp.dot(e, gv[:, sl], preferred_element_type=jnp.float32)
            den = jnp.dot(e, gones, preferred_element_type=jnp.float32)
            gohs.append(_bf(oh / den))
        gattn = jnp.concatenate(gohs, axis=1)
        go = _nt(gattn, gwop[...]) + gbo[...]
        rp = _layernorm(rr + go, gg[...], gb[...])
        vec = jnp.mean(rp, axis=0, keepdims=True)
        vec_ref[...] = jnp.broadcast_to(vec, (8, HIDDEN))

    @pl.when(g == 0)
    def _():
        qkv_into_slots(_bf(xs_a[...]), 0, 1)
        kr_ref[7] = jnp.zeros((128, HP), jnp.bfloat16)
        vr_ref[7] = jnp.zeros((128, HP), jnp.bfloat16)


def _add_body(refined, vec, h_ref):
    h_ref[...] = refined[...] + vec[0:1, :]


def _pad_params(Wqkv, bqkv, Wo):
    wq, wk, wv = jnp.split(Wqkv, 3, axis=0)

    def padw(w, scale=1.0):
        w3 = (w * scale).reshape(NHEADS, DH, HIDDEN)
        w3 = jnp.pad(w3, ((0, 0), (0, DP - DH), (0, 0)))
        return _bf(w3.reshape(HP, HIDDEN))

    bq, bk, bv = jnp.split(bqkv, 3)

    def padb(b, scale=1.0):
        b2 = (b * scale).reshape(NHEADS, DH)
        b2 = jnp.pad(b2, ((0, 0), (0, DP - DH)))
        return b2.reshape(1, HP).astype(jnp.float32)

    wqkvp = jnp.concatenate([padw(wq, SCALE), padw(wk), padw(wv)], axis=0)
    bqkvp = jnp.concatenate([padb(bq, SCALE), padb(bk), padb(bv)], axis=1)
    wot = Wo.T.reshape(NHEADS, DH, HIDDEN)
    wot = jnp.pad(wot, ((0, 0), (0, DP - DH), (0, 0))).reshape(HP, HIDDEN)
    wop = _bf(wot.T)
    return wqkvp, bqkvp, wop


def _tc_pipeline(xs2d, l_Wqkv, l_bqkv, l_Wo, l_bo, l_g, l_b,
                 g_Wqkv, g_bqkv, g_Wo, g_bo, g_g, g_b, interpret=False):
    lwqkvp, lbqkvp, lwop = _pad_params(l_Wqkv, l_bqkv, l_Wo)
    gwqkvp, gbqkvp, gwop = _pad_params(g_Wqkv, g_bqkv, g_Wo)
    lbo = l_bo.reshape(1, HIDDEN)
    lg = l_g.reshape(1, HIDDEN)
    lb = l_b.reshape(1, HIDDEN)
    gbo = g_bo.reshape(1, HIDDEN)
    gg = g_g.reshape(1, HIDDEN)
    gb = g_b.reshape(1, HIDDEN)

    full = lambda shp: pl.BlockSpec(shp, lambda i: (0,) * len(shp))
    blk = lambda shp, im: pl.BlockSpec(shp, im)

    refined, vec = pl.pallas_call(
        _local_body,
        grid=(NBLK // 2 + 1,),
        in_specs=[
            blk((256, HIDDEN), lambda i: (jnp.maximum(i - 1, 0), 0)),
            blk((256, HIDDEN), lambda i: (jnp.minimum(i + 1, NBLK // 2 - 1), 0)),
            full((3 * HP, HIDDEN)), full((1, 3 * HP)), full((HIDDEN, HP)),
            full((1, HIDDEN)), full((1, HIDDEN)), full((1, HIDDEN)),
            full((3 * HP, HIDDEN)), full((1, 3 * HP)), full((HIDDEN, HP)),
            full((1, HIDDEN)), full((1, HIDDEN)), full((1, HIDDEN)),
            blk((1, 256, 4 * 128), lambda i: (jnp.maximum(i - 1, 0), 0, 0)),
            blk((1, NWIN, 256), lambda i: (jnp.maximum(i - 1, 0), 0, 0)),
        ],
        out_specs=[
            blk((256, HIDDEN), lambda i: (jnp.maximum(i - 1, 0), 0)),
            full((8, HIDDEN)),
        ],
        out_shape=[
            jax.ShapeDtypeStruct((L, HIDDEN), jnp.float32),
            jax.ShapeDtypeStruct((8, HIDDEN), jnp.float32),
        ],
        scratch_shapes=[
            pltpu.VMEM((8, 128, HP), jnp.bfloat16),
            pltpu.VMEM((8, 128, HP), jnp.bfloat16),
            pltpu.VMEM((8, 128, HP), jnp.bfloat16),
            pltpu.VMEM((NWIN, HIDDEN), jnp.float32),
        ],
        interpret=interpret,
    )(xs2d, xs2d, lwqkvp, lbqkvp, lwop, lbo, lg, lb,
      gwqkvp, gbqkvp, gwop, gbo, gg, gb,
      jnp.asarray(_BIASROT_NP), jnp.asarray(_SMATT2_NP))

    ADD_ROWS = 512
    h2d = pl.pallas_call(
        _add_body,
        grid=(L // ADD_ROWS,),
        in_specs=[
            blk((ADD_ROWS, HIDDEN), lambda i: (i, 0)),
            full((8, HIDDEN)),
        ],
        out_specs=blk((ADD_ROWS, HIDDEN), lambda i: (i, 0)),
        out_shape=jax.ShapeDtypeStruct((L, HIDDEN), jnp.float32),
        interpret=interpret,
    )(refined, vec)

    return h2d


def kernel(x, coords, weight_params, l_Wqkv, l_bqkv, l_Wo, l_bo, l_g, l_b,
           g_Wqkv, g_bqkv, g_Wo, g_bo, g_g, g_b):
    del coords, weight_params
    x2d = x.reshape(L, HIDDEN)
    xs2d = _sc_gather(x2d, jnp.asarray(_PERM_NP))
    h2d = _tc_pipeline(xs2d, l_Wqkv, l_bqkv, l_Wo, l_bo, l_g, l_b,
                       g_Wqkv, g_bqkv, g_Wo, g_bo, g_g, g_b)
    return h2d.reshape(1, L, HIDDEN)


# speedup vs baseline: 2.1316x; 1.0209x over previous
import functools
import math

import jax
import jax.numpy as jnp
import numpy as np
from jax import lax
from jax.experimental import pallas as pl
from jax.experimental.pallas import tpu as pltpu
from jax.experimental.pallas import tpu_sc as plsc

HIDDEN = 384
NHEADS = 8
DH = HIDDEN // NHEADS
DP = 128
HP = NHEADS * DP
CLUSTER_SIZE = 64
L = 4096
NBLK = L // 128
SCALE = 1.0 / math.sqrt(DH)
EPS = 1e-5


def _static_plan():
    n_cluster = max(1, L // CLUSTER_SIZE)
    labels = np.random.RandomState(0).randint(0, n_cluster, size=L)
    index = np.argsort(labels, kind="stable")
    window_sizes = np.bincount(labels).tolist()
    new_sizes = []
    for size in window_sizes:
        if size >= CLUSTER_SIZE * 2:
            num_splits = max(1, size // CLUSTER_SIZE)
            q, r = divmod(size, num_splits)
            new_sizes.extend([q + 1 if i < r else q for i in range(num_splits)])
        else:
            new_sizes.append(size)
    sizes = [s for s in new_sizes if s > 0]
    return index.astype(np.int32), sizes


_PERM_NP, _SIZES = _static_plan()
NWIN = len(_SIZES)

_SEG_NP = np.repeat(np.arange(NWIN, dtype=np.int32), _SIZES)

_SEGQ_NP = _SEG_NP.reshape(NBLK, 128, 1)

_SEGK_NP = np.full((NBLK, 1, 3 * 128), -1, dtype=np.int32)
for _i in range(NBLK):
    _lo = (_i - 1) * 128
    _hi = (_i + 2) * 128
    _s = max(_lo, 0)
    _e = min(_hi, L)
    _SEGK_NP[_i, 0, _s - _lo:_e - _lo] = _SEG_NP[_s:_e]

_SMATT_NP = np.zeros((NBLK, NWIN, 128), dtype=np.float32)
for _i in range(NBLK):
    for _r in range(128):
        _w = _SEG_NP[_i * 128 + _r]
        _SMATT_NP[_i, _w, _r] = 1.0 / _SIZES[_w]

_BIASROT_NP = np.full((NBLK // 2, 256, 4 * 128), -1e30, dtype=np.float32)
for _B in range(NBLK // 2):
    _mq = _SEG_NP[_B * 256:(_B + 1) * 256][:, None]
    for _t, _c in enumerate(range(2 * _B - 1, 2 * _B + 3)):
        if 0 <= _c < NBLK:
            _mk = _SEG_NP[_c * 128:(_c + 1) * 128][None, :]
            _BIASROT_NP[_B][:, _t * 128:(_t + 1) * 128] = np.where(
                _mq == _mk, 0.0, -1e30)

_SMATT2_NP = np.zeros((NBLK // 2, NWIN, 256), dtype=np.float32)
for _B in range(NBLK // 2):
    for _r in range(256):
        _w = _SEG_NP[_B * 256 + _r]
        _SMATT2_NP[_B, _w, _r] = 1.0 / _SIZES[_w]


def _nt(a, b):
    return lax.dot_general(a, b, (((1,), (1,)), ((), ())),
                           preferred_element_type=jnp.float32)


def _bf(t):
    return t.astype(jnp.bfloat16)


def _sc_gather(x2d, idx):
    rows_per_w = L // 32
    mesh = plsc.VectorSubcoreMesh(core_axis_name="c", subcore_axis_name="s",
                                  num_cores=2, num_subcores=16)

    @functools.partial(
        pl.kernel,
        out_type=jax.ShapeDtypeStruct((L, HIDDEN), jnp.float32),
        mesh=mesh,
        scratch_types=[
            pltpu.VMEM((rows_per_w,), jnp.int32),
            pltpu.VMEM((rows_per_w, HIDDEN), jnp.float32),
            pltpu.SemaphoreType.DMA,
        ],
    )
    def body(x_hbm, idx_hbm, out_hbm, idx_v, rows_v, sem):
        wid = lax.axis_index("s") * 2 + lax.axis_index("c")
        base = wid * rows_per_w
        pltpu.sync_copy(idx_hbm.at[pl.ds(base, rows_per_w)], idx_v)
        pltpu.async_copy(x_hbm.at[idx_v], rows_v, sem).wait()
        pltpu.sync_copy(rows_v, out_hbm.at[pl.ds(base, rows_per_w)])

    return body(x2d, idx)


def _attend_padded(xq_bf, xkv_bf, wqp, wkp, wvp, bqp, bkp, bvp, wop, mask):
    q = _bf(_nt(xq_bf, wqp) + bqp)
    k = _bf(_nt(xkv_bf, wkp) + bkp)
    v = _bf(_nt(xkv_bf, wvp) + bvp)
    ones_bf = jnp.ones((xkv_bf.shape[0], DP), jnp.bfloat16)
    ohs = []
    for h in range(NHEADS):
        sl = slice(h * DP, (h + 1) * DP)
        s = _nt(q[:, sl], k[:, sl])
        if mask is not None:
            s = jnp.where(mask, s, -1e30)
        e = _bf(jnp.exp(s))
        oh = jnp.dot(e, v[:, sl], preferred_element_type=jnp.float32)
        den = jnp.dot(e, ones_bf, preferred_element_type=jnp.float32)
        ohs.append(_bf(oh / den))
    attn = jnp.concatenate(ohs, axis=1)
    return _nt(attn, wop)


def _layernorm(x, g, b):
    mu = jnp.mean(x, axis=1, keepdims=True)
    xc = x - mu
    var = jnp.mean(xc * xc, axis=1, keepdims=True)
    return xc * lax.rsqrt(var + EPS) * g + b


def _local_body(xs_a, xs_b, wqkvp, bqkvp, wop, bo, lg, lb,
                gwqkvp, gbqkvp, gwop, gbo, gg, gb, biasrot, smatt,
                refined_ref, vec_ref, qr_ref, kr_ref, vr_ref, acc_ref):
    g = pl.program_id(0)
    B = jnp.maximum(g - 1, 0)

    q_cur = jnp.concatenate(
        [qr_ref[lax.rem(2 * B, 8)], qr_ref[lax.rem(2 * B + 1, 8)]], axis=0)
    kslots = [lax.rem(2 * B - 1 + t + 8, 8) for t in range(4)]
    kall = jnp.concatenate([kr_ref[s] for s in kslots], axis=0)
    vall = jnp.concatenate([vr_ref[s] for s in kslots], axis=0)
    bias = biasrot[0]
    ones_bf = jnp.ones((4 * 128, DP), jnp.bfloat16)
    ohs = []
    for h in range(NHEADS):
        sl = slice(h * DP, (h + 1) * DP)
        s = _nt(q_cur[:, sl], kall[:, sl]) + bias
        e = _bf(jnp.exp(s))
        oh = jnp.dot(e, vall[:, sl], preferred_element_type=jnp.float32)
        den = jnp.dot(e, ones_bf, preferred_element_type=jnp.float32)
        ohs.append(_bf(oh / den))
    attn = jnp.concatenate(ohs, axis=1)
    o = _nt(attn, wop[...]) + bo[...]
    refined = _layernorm(xs_a[...] + o, lg[...], lb[...])
    refined_ref[...] = refined
    part = jnp.dot(smatt[0], _bf(refined), preferred_element_type=jnp.float32)

    wqkv = wqkvp[...]
    bqkv = bqkvp[...]

    def qkv_into_slots(x_bf, s0, s1):
        qkv = _nt(x_bf, wqkv) + bqkv
        qr_ref[s0] = _bf(qkv[:128, :HP])
        qr_ref[s1] = _bf(qkv[128:, :HP])
        kr_ref[s0] = _bf(qkv[:128, HP:2 * HP])
        kr_ref[s1] = _bf(qkv[128:, HP:2 * HP])
        vr_ref[s0] = _bf(qkv[:128, 2 * HP:])
        vr_ref[s1] = _bf(qkv[128:, 2 * HP:])

    qkv_into_slots(_bf(xs_b[...]), lax.rem(2 * g + 2, 8),
                   lax.rem(2 * g + 3, 8))

    b2 = g - 1

    @pl.when(b2 == 0)
    def _():
        acc_ref[...] = part

    @pl.when(b2 > 0)
    def _():
        acc_ref[...] = acc_ref[...] + part

    @pl.when(b2 == NBLK // 2 - 1)
    def _():
        rr = acc_ref[...]
        rr_bf = _bf(rr)
        gqkv = _nt(rr_bf, gwqkvp[...]) + gbqkvp[...]
        gq = _bf(gqkv[:, :HP])
        gk = _bf(gqkv[:, HP:2 * HP])
        gv = _bf(gqkv[:, 2 * HP:])
        gones = jnp.ones((NWIN, DP), jnp.bfloat16)
        gohs = []
        for h in range(NHEADS):
            sl = slice(h * DP, (h + 1) * DP)
            s = _nt(gq[:, sl], gk[:, sl])
            e = _bf(jnp.exp(s))
            oh = jnp.dot(e, gv[:, sl], preferred_element_type=jnp.float32)
            den = jnp.dot(e, gones, preferred_element_type=jnp.float32)
            gohs.append(_bf(oh / den))
        gattn = jnp.concatenate(gohs, axis=1)
        go = _nt(gattn, gwop[...]) + gbo[...]
        rp = _layernorm(rr + go, gg[...], gb[...])
        vec = jnp.mean(rp, axis=0, keepdims=True)
        vec_ref[...] = jnp.broadcast_to(vec, (8, HIDDEN))

    @pl.when(g == 0)
    def _():
        qkv_into_slots(_bf(xs_a[...]), 0, 1)
        kr_ref[7] = jnp.zeros((128, HP), jnp.bfloat16)
        vr_ref[7] = jnp.zeros((128, HP), jnp.bfloat16)


def _add_body(refined, vec, h_ref):
    h_ref[...] = refined[...] + vec[0:1, :]


def _pad_params(Wqkv, bqkv, Wo):
    wq, wk, wv = jnp.split(Wqkv, 3, axis=0)

    def padw(w, scale=1.0):
        w3 = (w * scale).reshape(NHEADS, DH, HIDDEN)
        w3 = jnp.pad(w3, ((0, 0), (0, DP - DH), (0, 0)))
        return _bf(w3.reshape(HP, HIDDEN))

    bq, bk, bv = jnp.split(bqkv, 3)

    def padb(b, scale=1.0):
        b2 = (b * scale).reshape(NHEADS, DH)
        b2 = jnp.pad(b2, ((0, 0), (0, DP - DH)))
        return b2.reshape(1, HP).astype(jnp.float32)

    wqkvp = jnp.concatenate([padw(wq, SCALE), padw(wk), padw(wv)], axis=0)
    bqkvp = jnp.concatenate([padb(bq, SCALE), padb(bk), padb(bv)], axis=1)
    wot = Wo.T.reshape(NHEADS, DH, HIDDEN)
    wot = jnp.pad(wot, ((0, 0), (0, DP - DH), (0, 0))).reshape(HP, HIDDEN)
    wop = _bf(wot.T)
    return wqkvp, bqkvp, wop


def _tc_pipeline(xs2d, l_Wqkv, l_bqkv, l_Wo, l_bo, l_g, l_b,
                 g_Wqkv, g_bqkv, g_Wo, g_bo, g_g, g_b, interpret=False):
    lwqkvp, lbqkvp, lwop = _pad_params(l_Wqkv, l_bqkv, l_Wo)
    gwqkvp, gbqkvp, gwop = _pad_params(g_Wqkv, g_bqkv, g_Wo)
    lbo = l_bo.reshape(1, HIDDEN)
    lg = l_g.reshape(1, HIDDEN)
    lb = l_b.reshape(1, HIDDEN)
    gbo = g_bo.reshape(1, HIDDEN)
    gg = g_g.reshape(1, HIDDEN)
    gb = g_b.reshape(1, HIDDEN)

    full = lambda shp: pl.BlockSpec(shp, lambda i: (0,) * len(shp))
    blk = lambda shp, im: pl.BlockSpec(shp, im)

    refined, vec = pl.pallas_call(
        _local_body,
        grid=(NBLK // 2 + 1,),
        in_specs=[
            blk((256, HIDDEN), lambda i: (jnp.maximum(i - 1, 0), 0)),
            blk((256, HIDDEN), lambda i: (jnp.minimum(i + 1, NBLK // 2 - 1), 0)),
            full((3 * HP, HIDDEN)), full((1, 3 * HP)), full((HIDDEN, HP)),
            full((1, HIDDEN)), full((1, HIDDEN)), full((1, HIDDEN)),
            full((3 * HP, HIDDEN)), full((1, 3 * HP)), full((HIDDEN, HP)),
            full((1, HIDDEN)), full((1, HIDDEN)), full((1, HIDDEN)),
            blk((1, 256, 4 * 128), lambda i: (jnp.maximum(i - 1, 0), 0, 0)),
            blk((1, NWIN, 256), lambda i: (jnp.maximum(i - 1, 0), 0, 0)),
        ],
        out_specs=[
            blk((256, HIDDEN), lambda i: (jnp.maximum(i - 1, 0), 0)),
            full((8, HIDDEN)),
        ],
        out_shape=[
            jax.ShapeDtypeStruct((L, HIDDEN), jnp.float32),
            jax.ShapeDtypeStruct((8, HIDDEN), jnp.float32),
        ],
        scratch_shapes=[
            pltpu.VMEM((8, 128, HP), jnp.bfloat16),
            pltpu.VMEM((8, 128, HP), jnp.bfloat16),
            pltpu.VMEM((8, 128, HP), jnp.bfloat16),
            pltpu.VMEM((NWIN, HIDDEN), jnp.float32),
        ],
        interpret=interpret,
    )(xs2d, xs2d, lwqkvp, lbqkvp, lwop, lbo, lg, lb,
      gwqkvp, gbqkvp, gwop, gbo, gg, gb,
      jnp.asarray(_BIASROT_NP), jnp.asarray(_SMATT2_NP, dtype=jnp.bfloat16))

    ADD_ROWS = 1024
    h2d = pl.pallas_call(
        _add_body,
        grid=(L // ADD_ROWS,),
        in_specs=[
            blk((ADD_ROWS, HIDDEN), lambda i: (i, 0)),
            full((8, HIDDEN)),
        ],
        out_specs=blk((ADD_ROWS, HIDDEN), lambda i: (i, 0)),
        out_shape=jax.ShapeDtypeStruct((L, HIDDEN), jnp.float32),
        interpret=interpret,
    )(refined, vec)

    return h2d


def kernel(x, coords, weight_params, l_Wqkv, l_bqkv, l_Wo, l_bo, l_g, l_b,
           g_Wqkv, g_bqkv, g_Wo, g_bo, g_g, g_b):
    del coords, weight_params
    x2d = x.reshape(L, HIDDEN)
    xs2d = _sc_gather(x2d, jnp.asarray(_PERM_NP))
    h2d = _tc_pipeline(xs2d, l_Wqkv, l_bqkv, l_Wo, l_bo, l_g, l_b,
                       g_Wqkv, g_bqkv, g_Wo, g_bo, g_g, g_b)
    return h2d.reshape(1, L, HIDDEN)
